# Initial kernel scaffold; baseline (speedup 1.0000x reference)
import jax
import jax.numpy as jnp
from jax.experimental import pallas as pl


def kernel(x, edge_index, batch, params):
    raise NotImplementedError("write your pallas kernel here")


# speedup vs baseline: 6.5933x; 6.5933x over previous
import functools

import jax
import jax.numpy as jnp
from jax import lax
from jax.experimental import pallas as pl
from jax.experimental.pallas import tpu as pltpu
from jax.experimental.pallas import tpu_sc as plsc

N = 10000
E = 320000
D = 128
G = 16
NC = 2
NS = 16
NW = NC * NS
EPT = E // NW
CH = 80
NCH = EPT // CH
RPT_STEP = 624
RPT_SZ = 640

def _sc_agg_body(h_hbm, src_hbm, dst_hbm, zero_hbm, out_hbm,
                 src_v, dst_v, rows, acc, sem):
    c = lax.axis_index("c")
    s = lax.axis_index("s")
    wid = c * NS + s
    pltpu.sync_copy(src_hbm.at[wid], src_v)
    pltpu.sync_copy(dst_hbm.at[wid], dst_v)
    pltpu.sync_copy(zero_hbm.at[pl.ds(s * RPT_STEP, RPT_SZ)],
                    acc.at[pl.ds(s * RPT_STEP, RPT_SZ)])
    plsc.subcore_barrier()

    def body(j, carry):
        cp = pltpu.async_copy(h_hbm.at[src_v.at[j]], rows, sem)
        cp.wait()
        pltpu.sync_copy(rows, acc.at[dst_v.at[j]], add=True)
        return carry

    lax.fori_loop(0, NCH, body, 0)
    plsc.subcore_barrier()
    pltpu.sync_copy(acc.at[pl.ds(s * RPT_STEP, RPT_SZ)],
                    out_hbm.at[c, pl.ds(s * RPT_STEP, RPT_SZ)])


@functools.cache
def _get_sc_agg():
    mesh = plsc.VectorSubcoreMesh(
        core_axis_name="c", subcore_axis_name="s",
        num_cores=NC, num_subcores=NS)
    return pl.kernel(
        _sc_agg_body,
        out_type=jax.ShapeDtypeStruct((NC, N, D), jnp.float32),
        mesh=mesh,
        scratch_types=[
            pltpu.VMEM((NCH, CH), jnp.int32),
            pltpu.VMEM((NCH, CH), jnp.int32),
            pltpu.VMEM((CH, D), jnp.float32),
            pltpu.VMEM_SHARED((N, D), jnp.float32),
            pltpu.SemaphoreType.DMA,
        ],
    )


def _dense_body(h_ref, a_ref, w1_ref, b1_ref, w2_ref, b2_ref, g_ref, bb_ref,
                o_ref, *, leaky):
    z = h_ref[:] + a_ref[0] + a_ref[1]
    z = jnp.dot(z, w1_ref[:], preferred_element_type=jnp.float32) + b1_ref[:]
    z = jnp.maximum(z, 0.0)
    z = jnp.dot(z, w2_ref[:], preferred_element_type=jnp.float32) + b2_ref[:]
    mu = jnp.mean(z, axis=0, keepdims=True)
    zc = z - mu
    var = jnp.mean(zc * zc, axis=0, keepdims=True)
    z = zc / jnp.sqrt(var + 1e-5) * g_ref[:] + bb_ref[:]
    if leaky:
        z = jnp.where(z >= 0.0, z, 0.01 * z)
    o_ref[:] = z


_tc_layer = pl.pallas_call(
    functools.partial(_dense_body, leaky=True),
    out_shape=jax.ShapeDtypeStruct((N, D), jnp.float32),
)


def _final_body(h_ref, a_ref, w1_ref, b1_ref, w2_ref, b2_ref, g_ref, bb_ref,
                batch_ref, wc_ref, bc_ref, o_ref):
    z = h_ref[:] + a_ref[0] + a_ref[1]
    z = jnp.dot(z, w1_ref[:], preferred_element_type=jnp.float32) + b1_ref[:]
    z = jnp.maximum(z, 0.0)
    z = jnp.dot(z, w2_ref[:], preferred_element_type=jnp.float32) + b2_ref[:]
    mu = jnp.mean(z, axis=0, keepdims=True)
    zc = z - mu
    var = jnp.mean(zc * zc, axis=0, keepdims=True)
    z = zc / jnp.sqrt(var + 1e-5) * g_ref[:] + bb_ref[:]
    onehot = (batch_ref[:] == lax.broadcasted_iota(jnp.int32, (N, G), 1)
              ).astype(jnp.float32)
    cnt = lax.dot_general(onehot, jnp.ones((N, 1), jnp.float32),
                          (((0,), (0,)), ((), ())),
                          preferred_element_type=jnp.float32,
                          precision=lax.Precision.HIGHEST)
    seg = lax.dot_general(onehot, z, (((0,), (0,)), ((), ())),
                          preferred_element_type=jnp.float32,
                          precision=lax.Precision.HIGHEST)
    pooled = seg / jnp.maximum(cnt, 1.0)
    o_ref[:] = jnp.dot(pooled, wc_ref[:],
                       preferred_element_type=jnp.float32) + bc_ref[:]


def kernel(x, edge_index, batch, params):
    src_r = edge_index[0].reshape(NW, NCH, CH)
    dst_r = edge_index[1].reshape(NW, NCH, CH)
    zeros = jnp.zeros((N, D), jnp.float32)
    batch2d = batch.reshape(N, 1)
    n_out3 = params['bc'].shape[0]

    tc_final = pl.pallas_call(
        _final_body,
        out_shape=jax.ShapeDtypeStruct((G, n_out3), jnp.float32),
    )

    sc_agg = _get_sc_agg()
    h = x
    for l in range(3):
        p = params[f'gin{l}']
        agg = sc_agg(h, src_r, dst_r, zeros)
        args = (h, agg, p['W1'], p['b1'].reshape(1, D), p['W2'],
                p['b2'].reshape(1, D), params[f'bn{l}_g'].reshape(1, D),
                params[f'bn{l}_b'].reshape(1, D))
        if l < 2:
            h = _tc_layer(*args)
        else:
            coords = tc_final(*args, batch2d, params['Wc'],
                              params['bc'].reshape(1, n_out3))
    return coords.reshape(-1, 3)
